# baseline (speedup 1.0000x reference)
import jax
import jax.numpy as jnp
from jax.experimental import pallas as pl
from jax.experimental.pallas import tpu as pltpu

_N = 10000
_D = 128
_S = 5
_BH = 80
_STEPS = _N // (_S * _BH)
_RPS = _N // _S


def _gcn_body(a0, a1, a2, a3, a4, x_ref, w_ref, b_ref, al_ref, o_ref, xbf_s, wbf_s):
    i = pl.program_id(0)

    @pl.when(i == 0)
    def _prep():
        xbf_s[...] = x_ref[...].astype(jnp.bfloat16)
        wbf_s[...] = w_ref[...].astype(jnp.bfloat16)

    parts = [jnp.dot(r[...].astype(jnp.bfloat16), xbf_s[...],
                     preferred_element_type=jnp.float32)
             for r in (a0, a1, a2, a3, a4)]
    ax = jnp.concatenate(parts, axis=0)
    xblk = jnp.concatenate(
        [x_ref[pl.ds(s * _RPS + i * _BH, _BH), :] for s in range(_S)],
        axis=0)
    axw = jnp.dot(ax.astype(jnp.bfloat16), wbf_s[...],
                  preferred_element_type=jnp.float32)
    l0 = al_ref[0, 0]
    l1 = al_ref[0, 1]
    m = jnp.maximum(l0, l1)
    e0 = jnp.exp(l0 - m)
    e1 = jnp.exp(l1 - m)
    g0 = e0 / (e0 + e1)
    g1 = e1 / (e0 + e1)
    y = g0 * axw + g1 * xblk + b_ref[...]
    y = jnp.where(y > 0.0, y, jnp.exp(jnp.minimum(y, 0.0)) - 1.0)
    o_ref[...] = y.reshape(_S, 1, _BH, _D)


def kernel(inputs, adj, weight, bias, alpha):
    bias2 = bias.reshape(1, _D)
    al2 = alpha.reshape(1, 2)

    def _a_spec(s):
        return pl.BlockSpec((_BH, _N), lambda i, s=s: (s * _STEPS + i, 0))

    out4d = pl.pallas_call(
        _gcn_body,
        grid=(_STEPS,),
        in_specs=[
            _a_spec(0), _a_spec(1), _a_spec(2), _a_spec(3), _a_spec(4),
            pl.BlockSpec((_N, _D), lambda i: (0, 0)),
            pl.BlockSpec((_D, _D), lambda i: (0, 0)),
            pl.BlockSpec((1, _D), lambda i: (0, 0)),
            pl.BlockSpec((1, 2), lambda i: (0, 0)),
        ],
        out_specs=pl.BlockSpec((_S, 1, _BH, _D), lambda i: (0, i, 0, 0)),
        out_shape=jax.ShapeDtypeStruct((_S, _STEPS, _BH, _D), jnp.float32),
        scratch_shapes=[
            pltpu.VMEM((_N, _D), jnp.bfloat16),
            pltpu.VMEM((_D, _D), jnp.bfloat16),
        ],
        compiler_params=pltpu.CompilerParams(
            dimension_semantics=("arbitrary",),
        ),
    )(adj, adj, adj, adj, adj, inputs, weight, bias2, al2)
    return out4d.reshape(_N, _D)


# speedup vs baseline: 1.0036x; 1.0036x over previous
import jax
import jax.numpy as jnp
from jax.experimental import pallas as pl
from jax.experimental.pallas import tpu as pltpu

_N = 10000
_D = 128
_S = 2
_BH = 200
_STEPS = _N // (_S * _BH)
_RPS = _N // _S


def _gcn_body(a0, a1, x_ref, w_ref, b_ref, al_ref, o_ref, sup_s):
    i = pl.program_id(0)

    @pl.when(i == 0)
    def _prep():
        xw = jnp.dot(x_ref[...].astype(jnp.bfloat16),
                     w_ref[...].astype(jnp.bfloat16),
                     preferred_element_type=jnp.float32)
        sup_s[...] = xw.astype(jnp.bfloat16)

    parts = [jnp.dot(r[...].astype(jnp.bfloat16), sup_s[...],
                     preferred_element_type=jnp.float32)
             for r in (a0, a1)]
    asup = jnp.concatenate(parts, axis=0)
    xblk = jnp.concatenate(
        [x_ref[pl.ds(s * _RPS + i * _BH, _BH), :] for s in range(_S)],
        axis=0)
    l0 = al_ref[0, 0]
    l1 = al_ref[0, 1]
    m = jnp.maximum(l0, l1)
    e0 = jnp.exp(l0 - m)
    e1 = jnp.exp(l1 - m)
    g0 = e0 / (e0 + e1)
    g1 = e1 / (e0 + e1)
    y = g0 * asup + g1 * xblk + b_ref[...]
    y = jnp.where(y > 0.0, y, jnp.exp(jnp.minimum(y, 0.0)) - 1.0)
    o_ref[...] = y.reshape(_S, 1, _BH, _D)


def kernel(inputs, adj, weight, bias, alpha):
    bias2 = bias.reshape(1, _D)
    al2 = alpha.reshape(1, 2)

    def _a_spec(s):
        return pl.BlockSpec((_BH, _N), lambda i, s=s: (s * _STEPS + i, 0))

    out4d = pl.pallas_call(
        _gcn_body,
        grid=(_STEPS,),
        in_specs=[
            _a_spec(0), _a_spec(1),
            pl.BlockSpec((_N, _D), lambda i: (0, 0)),
            pl.BlockSpec((_D, _D), lambda i: (0, 0)),
            pl.BlockSpec((1, _D), lambda i: (0, 0)),
            pl.BlockSpec((1, 2), lambda i: (0, 0)),
        ],
        out_specs=pl.BlockSpec((_S, 1, _BH, _D), lambda i: (0, i, 0, 0)),
        out_shape=jax.ShapeDtypeStruct((_S, _STEPS, _BH, _D), jnp.float32),
        scratch_shapes=[
            pltpu.VMEM((_N, _D), jnp.bfloat16),
        ],
        compiler_params=pltpu.CompilerParams(
            dimension_semantics=("arbitrary",),
        ),
    )(adj, adj, inputs, weight, bias2, al2)
    return out4d.reshape(_N, _D)


# speedup vs baseline: 1.0106x; 1.0070x over previous
import jax
import jax.numpy as jnp
from jax.experimental import pallas as pl
from jax.experimental.pallas import tpu as pltpu

_N = 10000
_D = 128
_S = 1
_BH = 400
_STEPS = _N // (_S * _BH)
_RPS = _N // _S


def _gcn_body(a0, x_ref, w_ref, b_ref, al_ref, o_ref, sup_s):
    i = pl.program_id(0)

    @pl.when(i == 0)
    def _prep():
        xw = jnp.dot(x_ref[...].astype(jnp.bfloat16),
                     w_ref[...].astype(jnp.bfloat16),
                     preferred_element_type=jnp.float32)
        sup_s[...] = xw.astype(jnp.bfloat16)

    parts = [jnp.dot(r[...].astype(jnp.bfloat16), sup_s[...],
                     preferred_element_type=jnp.float32)
             for r in (a0,)]
    asup = jnp.concatenate(parts, axis=0)
    xblk = jnp.concatenate(
        [x_ref[pl.ds(s * _RPS + i * _BH, _BH), :] for s in range(_S)],
        axis=0)
    l0 = al_ref[0, 0]
    l1 = al_ref[0, 1]
    m = jnp.maximum(l0, l1)
    e0 = jnp.exp(l0 - m)
    e1 = jnp.exp(l1 - m)
    g0 = e0 / (e0 + e1)
    g1 = e1 / (e0 + e1)
    y = g0 * asup + g1 * xblk + b_ref[...]
    y = jnp.where(y > 0.0, y, jnp.exp(jnp.minimum(y, 0.0)) - 1.0)
    o_ref[...] = y.reshape(_S, 1, _BH, _D)


def kernel(inputs, adj, weight, bias, alpha):
    bias2 = bias.reshape(1, _D)
    al2 = alpha.reshape(1, 2)

    def _a_spec(s):
        return pl.BlockSpec((_BH, _N), lambda i, s=s: (s * _STEPS + i, 0))

    out4d = pl.pallas_call(
        _gcn_body,
        grid=(_STEPS,),
        in_specs=[
            _a_spec(0),
            pl.BlockSpec((_N, _D), lambda i: (0, 0)),
            pl.BlockSpec((_D, _D), lambda i: (0, 0)),
            pl.BlockSpec((1, _D), lambda i: (0, 0)),
            pl.BlockSpec((1, 2), lambda i: (0, 0)),
        ],
        out_specs=pl.BlockSpec((_S, 1, _BH, _D), lambda i: (0, i, 0, 0)),
        out_shape=jax.ShapeDtypeStruct((_S, _STEPS, _BH, _D), jnp.float32),
        scratch_shapes=[
            pltpu.VMEM((_N, _D), jnp.bfloat16),
        ],
        compiler_params=pltpu.CompilerParams(
            dimension_semantics=("arbitrary",),
        ),
    )(adj, inputs, weight, bias2, al2)
    return out4d.reshape(_N, _D)
